# Initial kernel scaffold; baseline (speedup 1.0000x reference)
import jax
import jax.numpy as jnp
from jax.experimental import pallas as pl


def kernel(x, edge_index, edge_attr, Wl1, Wr1, att1, b1, Wl2, Wr2, att2, b2, W_ih, W_hh, b_ih, b_hh):
    raise NotImplementedError("write your pallas kernel here")


# speedup vs baseline: 2.6918x; 2.6918x over previous
import functools

import jax
import jax.numpy as jnp
from jax.experimental import pallas as pl
from jax.experimental.pallas import tpu as pltpu


def _ih_body(x_ref, w_ref, b_ref, o_ref):
    o_ref[0] = jax.lax.dot_general(
        x_ref[...], w_ref[0], (((1,), (1,)), ((), ())),
        preferred_element_type=jnp.float32) + b_ref[0]


def _lstm_ih(x_all, W4, b4):
    TB, H = x_all.shape
    return pl.pallas_call(
        _ih_body,
        grid=(4,),
        in_specs=[
            pl.BlockSpec((TB, H), lambda g: (0, 0)),
            pl.BlockSpec((1, H, H), lambda g: (g, 0, 0)),
            pl.BlockSpec((1, 1, H), lambda g: (g, 0, 0)),
        ],
        out_specs=pl.BlockSpec((1, TB, H), lambda g: (g, 0, 0)),
        out_shape=jax.ShapeDtypeStruct((4, TB, H), jnp.float32),
    )(x_all, W4, b4)


def _rec_body(ih_ref, w_ref, c_ref, h_ref, h_scr, c_scr, g_scr):
    t = pl.program_id(0)
    g = pl.program_id(1)

    @pl.when((t == 0) & (g == 0))
    def _init():
        h_scr[...] = jnp.zeros_like(h_scr)
        c_scr[...] = jnp.zeros_like(c_scr)

    h = h_scr[...].astype(jnp.bfloat16)
    g_scr[g] = ih_ref[0, 0] + jax.lax.dot_general(
        h, w_ref[0], (((1,), (1,)), ((), ())),
        preferred_element_type=jnp.float32)

    @pl.when(g == 3)
    def _update():
        ig, fg, gg, og = g_scr[0], g_scr[1], g_scr[2], g_scr[3]
        c = (jax.nn.sigmoid(fg) * c_scr[...]
             + jax.nn.sigmoid(ig) * jnp.tanh(gg))
        hn = jax.nn.sigmoid(og) * jnp.tanh(c)
        c_scr[...] = c
        h_scr[...] = hn

        @pl.when(t == pl.num_programs(0) - 1)
        def _emit():
            c_ref[...] = c
            h_ref[...] = hn


def _lstm_rec(ih4, W4, B, H, T):
    out_sds = jax.ShapeDtypeStruct((B, H), jnp.float32)
    return pl.pallas_call(
        _rec_body,
        grid=(T, 4),
        in_specs=[
            pl.BlockSpec((1, 1, B, H), lambda t, g: (g, t, 0, 0)),
            pl.BlockSpec((1, H, H), lambda t, g: (g, 0, 0)),
        ],
        out_specs=[
            pl.BlockSpec((B, H), lambda t, g: (0, 0)),
            pl.BlockSpec((B, H), lambda t, g: (0, 0)),
        ],
        out_shape=[out_sds, out_sds],
        scratch_shapes=[
            pltpu.VMEM((B, H), jnp.float32),
            pltpu.VMEM((B, H), jnp.float32),
            pltpu.VMEM((4, B, H), jnp.float32),
        ],
    )(ih4, W4)


def _gat(x, src, dst, Wl, Wr, att, bias, N):
    xl = x @ Wl
    xr = x @ Wr
    e = jax.nn.leaky_relu(xl[src] + xr[dst], 0.2)
    logits = jnp.sum(e * att, axis=-1)
    ex = jnp.exp(logits)
    denom = jax.ops.segment_sum(ex, dst, num_segments=N)
    s1 = jax.ops.segment_sum(ex[:, None] * xl[src], dst, num_segments=N)
    return s1 / denom[:, None] + bias


def kernel(x, edge_index, edge_attr, Wl1, Wr1, att1, b1, Wl2, Wr2, att2, b2,
           W_ih, W_hh, b_ih, b_hh):
    batch, T, nodes = x.shape
    N = batch * nodes
    H = W_hh.shape[1]
    loops = jnp.arange(N, dtype=edge_index.dtype)
    src = jnp.concatenate([edge_index[0], loops])
    dst = jnp.concatenate([edge_index[1], loops])

    feats = []
    for t in range(T):
        h = x[:, t, :].reshape(-1, 1)
        h = jax.nn.relu(_gat(h, src, dst, Wl1, Wr1, att1, b1, N))
        h = jax.nn.relu(_gat(h, src, dst, Wl2, Wr2, att2, b2, N))
        feats.append(h.reshape(batch, H))
    x_all = jnp.stack(feats, axis=0).reshape(T * batch, H)

    W_ih4 = W_ih.reshape(4, H, H).astype(jnp.bfloat16)
    W_hh4 = W_hh.reshape(4, H, H).astype(jnp.bfloat16)
    b4 = (b_ih + b_hh).reshape(4, 1, H)

    ih4 = _lstm_ih(x_all.astype(jnp.bfloat16), W_ih4, b4)
    ih4 = ih4.reshape(4, T, batch, H)
    c_t, h_t = _lstm_rec(ih4, W_hh4, batch, H, T)
    return (c_t, h_t)
